# baseline (speedup 1.0000x reference)
import jax
import jax.numpy as jnp
import numpy as np
from jax import lax
from jax.experimental import pallas as pl
from jax.experimental.pallas import tpu as pltpu
from jax.experimental.pallas import tpu_sc as plsc

N = 2048
B = 4
K = 16
FD = 6

_PREC = lax.Precision.HIGHEST


def _dot(a, bm):
    return jnp.dot(a, bm, preferred_element_type=jnp.float32, precision=_PREC)


def _relu(v):
    return jnp.maximum(v, 0.0)


_CHUNK = 2048
_NCH = N // _CHUNK
_BF = jnp.bfloat16


def _split(v):
    hi = v.astype(_BF)
    lo = (v - hi.astype(jnp.float32)).astype(_BF)
    return hi, lo


def _bdot(a, bm):
    return jnp.dot(a, bm, preferred_element_type=jnp.float32)


_BLK = 512
_NBLK = N // _BLK


def _rhs_cat(src_ref, cin):
    hh, hl = _split(src_ref[:, :cin])
    return jnp.concatenate(
        [jnp.concatenate([hh, hl], axis=1),
         jnp.concatenate([hh, jnp.zeros_like(hl)], axis=1)], axis=0)


def _mm_chunk(mcat, rhs, cin, W, bias):
    p = _bdot(mcat, rhs)
    t = p[:, :cin] + p[:, cin:2 * cin]
    return _relu(_dot(t, W) + bias)


def _layer(src_ref, cin, cout, Wr, Br, dst_ref, cat_ref):
    W = Wr[...]
    bias = Br[...]
    rhs = _rhs_cat(src_ref, cin)

    def chunk(i, carry):
        off = i * _CHUNK
        mcat = cat_ref[pl.ds(off, _CHUNK), :]
        dst_ref[pl.ds(off, _CHUNK), :cout] = _mm_chunk(mcat, rhs, cin, W,
                                                       bias)
        return carry

    lax.fori_loop(0, _NCH, chunk, jnp.int32(0))


def _chain_body(L_ref, A_ref, x2_ref, b2_ref, W0, B0, W1, B1, W2, B2, W3, B3,
                feat_ref, h_sc, t_sc, cat_sc):
    i = pl.program_id(0)
    off = i * _BLK
    x2 = x2_ref[...]

    @pl.when(i == 0)
    def _():
        h_sc[:, 0:4] = x2
        h_sc[:, 8:12] = b2_ref[...]

    xh, xl = _split(x2)
    ahi, alo = _split(A_ref[...])
    h_sc[pl.ds(off, _BLK), 4:8] = (
        _bdot(ahi, xh) + _bdot(ahi, xl) + _bdot(alo, xh))
    mhi, mlo = _split(L_ref[...])
    cat_sc[pl.ds(off, _BLK), 0:N] = mhi
    cat_sc[pl.ds(off, _BLK), N:2 * N] = mlo

    @pl.when(i == _NBLK - 1)
    def _():
        _layer(h_sc, 12, 32, W0, B0, t_sc, cat_sc)
        _layer(t_sc, 32, 64, W1, B1, h_sc, cat_sc)
        _layer(h_sc, 64, 32, W2, B2, t_sc, cat_sc)
        _layer(t_sc, 32, 24, W3, B3, h_sc, cat_sc)
        feat_ref[...] = h_sc[:, :24]


def _ms_body(L_ref, feat_ref, W0, B0, W1, B1, out_ref, g_sc, t_sc, cat_sc):
    i = pl.program_id(0)
    off = i * _BLK

    @pl.when(i == 0)
    def _():
        g_sc[:, :24] = feat_ref[...]

    mhi, mlo = _split(L_ref[...])
    cat_sc[pl.ds(off, _BLK), 0:N] = mhi
    cat_sc[pl.ds(off, _BLK), N:2 * N] = mlo
    rhs1 = _rhs_cat(g_sc, 24)
    t_sc[pl.ds(off, _BLK), :24] = _mm_chunk(
        jnp.concatenate([mhi, mlo], axis=1), rhs1, 24, W0[...], B0[...])

    @pl.when(i == _NBLK - 1)
    def _():
        _layer(t_sc, 24, 24, W1, B1, g_sc, cat_sc)
        out_ref[...] = g_sc[:, :24]


_NPW = N // 32


def _attn_body(gh, knnh, outh, g2_v, knn_v, out_v):
    cid = lax.axis_index("c")
    sid = lax.axis_index("s")
    wid = sid * 2 + cid
    base = wid * _NPW

    pltpu.sync_copy(gh, g2_v)
    pltpu.sync_copy(knnh.at[pl.ds(base * K, _NPW * K)], knn_v)

    iota16 = lax.iota(jnp.int32, 16)
    inv_sqrt = np.float32(1.0 / np.sqrt(float(FD)))
    CB = FD * B
    nj = _NPW // 16

    def body(i, carry):
        bb = i // nj
        j = i % nj
        lrow = j * 16 + iota16
        grow = base + lrow
        cols = [cc * 4 + bb for cc in range(FD)]
        h = [plsc.load_gather(g2_v, [grow * CB + cols[cc]])
             for cc in range(FD)]
        scr = []
        for k in range(K):
            idx = plsc.load_gather(knn_v, [lrow * K + k])
            nbase = idx * CB
            s_k = h[0] * plsc.load_gather(g2_v, [nbase + cols[0]])
            for cc in range(1, FD):
                s_k = s_k + h[cc] * plsc.load_gather(g2_v, [nbase + cols[cc]])
            scr.append(s_k * inv_sqrt)
        m = scr[0]
        for k in range(1, K):
            m = jnp.maximum(m, scr[k])
        ek = [jnp.exp(scr[k] - m) for k in range(K)]
        ssum = ek[0]
        for k in range(1, K):
            ssum = ssum + ek[k]
        inv = 1.0 / ssum
        att = [ek[k] * inv for k in range(K)]
        agg = [jnp.zeros((16,), jnp.float32) for _ in range(FD)]
        for k in range(K):
            idx = plsc.load_gather(knn_v, [lrow * K + k])
            nbase = idx * CB
            for cc in range(FD):
                nb = plsc.load_gather(g2_v, [nbase + cols[cc]])
                agg[cc] = agg[cc] + att[k] * nb
        for cc in range(FD):
            plsc.store_scatter(out_v, [lrow * CB + cols[cc]], agg[cc])
        return carry

    lax.fori_loop(0, B * nj, body, jnp.int32(0))
    pltpu.sync_copy(out_v, outh.at[wid])


def _final_body(a0, a1, a2, a3, sens_ref, feat_ref, x2_ref, Wp1r, bp1r,
                Wp2r, bp2r, Wur, bur, out_ref):
    sv = sens_ref[...]
    e = jnp.exp(sv - jnp.max(sv))
    w = e / jnp.sum(e)
    fused = w[0] * a0[...] + w[1] * a1[...] + w[2] * a2[...] + w[3] * a3[...]
    t1 = _relu(_dot(fused, Wp1r[...]) + bp1r[...])
    grad = _dot(t1, Wp2r[...]) + bp2r[...]
    alpha = jax.nn.sigmoid(_dot(feat_ref[...], Wur[...]) + bur[...])
    out_ref[...] = x2_ref[...] - alpha * grad


def kernel(x, b, L, A, L0, L1, L2, L3, knn_idx, sens_w, Wg0, bg0, Wg1, bg1,
           Wg2, bg2, Wg3, bg3, Wms0, bms0, Wms1, bms1, Wp1, bp1, Wp2, bp2,
           Wu, bu):
    f32 = jnp.float32
    eye = jnp.eye(B, dtype=f32)
    x2 = x[:, :, 0].T
    b2 = b[:, :, 0].T

    def kr(W):
        return jnp.kron(W.astype(f32), eye)

    def rep(v):
        return jnp.repeat(v.astype(f32), B)

    def _whole(a):
        return pl.BlockSpec(a.shape, lambda i: (0,) * a.ndim)

    def _rows(a):
        return pl.BlockSpec((_BLK, N), lambda i: (i, 0))

    wspecs_a = [kr(Wg0), rep(bg0), kr(Wg1), rep(bg1),
                kr(Wg2), rep(bg2), kr(Wg3), rep(bg3)]
    feat = pl.pallas_call(
        _chain_body,
        grid=(_NBLK,),
        in_specs=[_rows(L), _rows(A), _whole(x2), _whole(b2)]
                 + [_whole(w) for w in wspecs_a],
        out_specs=pl.BlockSpec((N, FD * B), lambda i: (0, 0)),
        out_shape=jax.ShapeDtypeStruct((N, FD * B), f32),
        scratch_shapes=[pltpu.VMEM((N, 64), f32), pltpu.VMEM((N, 64), f32),
                        pltpu.VMEM((N, 2 * N), jnp.bfloat16)],
    )(L, A, x2, b2, *wspecs_a)

    def ms(Ls, ft, W0, B0, W1, B1):
        return pl.pallas_call(
            _ms_body,
            grid=(_NBLK,),
            in_specs=[_rows(Ls), _whole(ft), _whole(W0), _whole(B0),
                      _whole(W1), _whole(B1)],
            out_specs=pl.BlockSpec((N, FD * B), lambda i: (0, 0)),
            out_shape=jax.ShapeDtypeStruct((N, FD * B), f32),
            scratch_shapes=[pltpu.VMEM((N, 24), f32),
                            pltpu.VMEM((N, 24), f32),
                            pltpu.VMEM((N, 2 * N), jnp.bfloat16)],
        )(Ls, ft, W0, B0, W1, B1)
    g2s = [ms(Ls, feat, kr(Wms0[s]), rep(bms0[s]), kr(Wms1[s]), rep(bms1[s]))
           for s, Ls in enumerate((L0, L1, L2, L3))]

    attn = pl.kernel(
        _attn_body,
        mesh=plsc.VectorSubcoreMesh(core_axis_name="c", subcore_axis_name="s"),
        compiler_params=pltpu.CompilerParams(needs_layout_passes=False),
        out_type=jax.ShapeDtypeStruct((32, _NPW * FD * B), f32),
        scratch_types=[
            pltpu.VMEM((N * FD * B,), f32),
            pltpu.VMEM((_NPW * K,), jnp.int32),
            pltpu.VMEM((_NPW * FD * B,), f32),
        ],
    )
    knn_flat = knn_idx.astype(jnp.int32).reshape(-1)
    aggs = [attn(g.reshape(-1), knn_flat).reshape(N, FD * B) for g in g2s]

    out2 = pl.pallas_call(
        _final_body,
        out_shape=jax.ShapeDtypeStruct((N, B), f32),
    )(aggs[0], aggs[1], aggs[2], aggs[3], sens_w.astype(f32), feat, x2,
      kr(Wp1), rep(bp1), kr(Wp2), rep(bp2), kr(Wu), rep(bu))

    return out2.T.reshape(B, N, 1)


# speedup vs baseline: 1.0535x; 1.0535x over previous
import jax
import jax.numpy as jnp
import numpy as np
from jax import lax
from jax.experimental import pallas as pl
from jax.experimental.pallas import tpu as pltpu
from jax.experimental.pallas import tpu_sc as plsc

N = 2048
B = 4
K = 16
FD = 6

_PREC = lax.Precision.HIGHEST


def _dot(a, bm):
    return jnp.dot(a, bm, preferred_element_type=jnp.float32, precision=_PREC)


def _relu(v):
    return jnp.maximum(v, 0.0)


_CHUNK = 1024
_NCH = N // _CHUNK
_BF = jnp.bfloat16


def _split(v):
    hi = v.astype(_BF)
    lo = (v - hi.astype(jnp.float32)).astype(_BF)
    return hi, lo


def _bdot(a, bm):
    return jnp.dot(a, bm, preferred_element_type=jnp.float32)


_BLK = 256
_NBLK = N // _BLK


def _rhs_cat(src_ref, cin):
    hh, hl = _split(src_ref[:, :cin])
    return jnp.concatenate(
        [jnp.concatenate([hh, hl], axis=1),
         jnp.concatenate([hh, jnp.zeros_like(hl)], axis=1)], axis=0)


def _mm_chunk(mcat, rhs, cin, W, bias):
    p = _bdot(mcat, rhs)
    t = p[:, :cin] + p[:, cin:2 * cin]
    return _relu(_dot(t, W) + bias)


def _layer(src_ref, cin, cout, Wr, Br, dst_ref, cat_ref):
    W = Wr[...]
    bias = Br[...]
    rhs = _rhs_cat(src_ref, cin)

    def chunk(i, carry):
        off = i * _CHUNK
        mcat = cat_ref[pl.ds(off, _CHUNK), :]
        dst_ref[pl.ds(off, _CHUNK), :cout] = _mm_chunk(mcat, rhs, cin, W,
                                                       bias)
        return carry

    lax.fori_loop(0, _NCH, chunk, jnp.int32(0))


def _chain_body(L_ref, A_ref, x2_ref, b2_ref, W0, B0, W1, B1, W2, B2, W3, B3,
                feat_ref, h_sc, t_sc, cat_sc):
    i = pl.program_id(0)
    off = i * _BLK
    x2 = x2_ref[...]

    @pl.when(i == 0)
    def _():
        h_sc[:, 0:4] = x2
        h_sc[:, 8:12] = b2_ref[...]

    xh, xl = _split(x2)
    ahi, alo = _split(A_ref[...])
    h_sc[pl.ds(off, _BLK), 4:8] = (
        _bdot(ahi, xh) + _bdot(ahi, xl) + _bdot(alo, xh))
    mhi, mlo = _split(L_ref[...])
    cat_sc[pl.ds(off, _BLK), 0:N] = mhi
    cat_sc[pl.ds(off, _BLK), N:2 * N] = mlo

    @pl.when(i == _NBLK - 1)
    def _():
        _layer(h_sc, 12, 32, W0, B0, t_sc, cat_sc)
        _layer(t_sc, 32, 64, W1, B1, h_sc, cat_sc)
        _layer(h_sc, 64, 32, W2, B2, t_sc, cat_sc)
        _layer(t_sc, 32, 24, W3, B3, h_sc, cat_sc)
        feat_ref[...] = h_sc[:, :24]


def _ms_body(L_ref, feat_ref, W0, B0, W1, B1, out_ref, g_sc, t_sc, cat_sc):
    i = pl.program_id(0)
    off = i * _BLK

    @pl.when(i == 0)
    def _():
        g_sc[:, :24] = feat_ref[...]

    mhi, mlo = _split(L_ref[...])
    cat_sc[pl.ds(off, _BLK), 0:N] = mhi
    cat_sc[pl.ds(off, _BLK), N:2 * N] = mlo
    rhs1 = _rhs_cat(g_sc, 24)
    t_sc[pl.ds(off, _BLK), :24] = _mm_chunk(
        jnp.concatenate([mhi, mlo], axis=1), rhs1, 24, W0[...], B0[...])

    @pl.when(i == _NBLK - 1)
    def _():
        _layer(t_sc, 24, 24, W1, B1, g_sc, cat_sc)
        out_ref[...] = g_sc[:, :24]


_NPW = N // 32


def _attn_body(gh, knnh, outh, g2_v, knn_v, out_v):
    cid = lax.axis_index("c")
    sid = lax.axis_index("s")
    wid = sid * 2 + cid
    base = wid * _NPW

    pltpu.sync_copy(gh, g2_v)
    pltpu.sync_copy(knnh.at[pl.ds(base * K, _NPW * K)], knn_v)

    iota16 = lax.iota(jnp.int32, 16)
    inv_sqrt = np.float32(1.0 / np.sqrt(float(FD)))
    CB = FD * B
    nj = _NPW // 16

    def body(i, carry):
        bb = i // nj
        j = i % nj
        lrow = j * 16 + iota16
        grow = base + lrow
        cols = [cc * 4 + bb for cc in range(FD)]
        h = [plsc.load_gather(g2_v, [grow * CB + cols[cc]])
             for cc in range(FD)]
        scr = []
        for k in range(K):
            idx = plsc.load_gather(knn_v, [lrow * K + k])
            nbase = idx * CB
            s_k = h[0] * plsc.load_gather(g2_v, [nbase + cols[0]])
            for cc in range(1, FD):
                s_k = s_k + h[cc] * plsc.load_gather(g2_v, [nbase + cols[cc]])
            scr.append(s_k * inv_sqrt)
        m = scr[0]
        for k in range(1, K):
            m = jnp.maximum(m, scr[k])
        ek = [jnp.exp(scr[k] - m) for k in range(K)]
        ssum = ek[0]
        for k in range(1, K):
            ssum = ssum + ek[k]
        inv = 1.0 / ssum
        att = [ek[k] * inv for k in range(K)]
        agg = [jnp.zeros((16,), jnp.float32) for _ in range(FD)]
        for k in range(K):
            idx = plsc.load_gather(knn_v, [lrow * K + k])
            nbase = idx * CB
            for cc in range(FD):
                nb = plsc.load_gather(g2_v, [nbase + cols[cc]])
                agg[cc] = agg[cc] + att[k] * nb
        for cc in range(FD):
            plsc.store_scatter(out_v, [lrow * CB + cols[cc]], agg[cc])
        return carry

    lax.fori_loop(0, B * nj, body, jnp.int32(0))
    pltpu.sync_copy(out_v, outh.at[wid])


def _final_body(a0, a1, a2, a3, sens_ref, feat_ref, x2_ref, Wp1r, bp1r,
                Wp2r, bp2r, Wur, bur, out_ref):
    sv = sens_ref[...]
    e = jnp.exp(sv - jnp.max(sv))
    w = e / jnp.sum(e)
    fused = w[0] * a0[...] + w[1] * a1[...] + w[2] * a2[...] + w[3] * a3[...]
    t1 = _relu(_dot(fused, Wp1r[...]) + bp1r[...])
    grad = _dot(t1, Wp2r[...]) + bp2r[...]
    alpha = jax.nn.sigmoid(_dot(feat_ref[...], Wur[...]) + bur[...])
    out_ref[...] = x2_ref[...] - alpha * grad


def kernel(x, b, L, A, L0, L1, L2, L3, knn_idx, sens_w, Wg0, bg0, Wg1, bg1,
           Wg2, bg2, Wg3, bg3, Wms0, bms0, Wms1, bms1, Wp1, bp1, Wp2, bp2,
           Wu, bu):
    f32 = jnp.float32
    eye = jnp.eye(B, dtype=f32)
    x2 = x[:, :, 0].T
    b2 = b[:, :, 0].T

    def kr(W):
        return jnp.kron(W.astype(f32), eye)

    def rep(v):
        return jnp.repeat(v.astype(f32), B)

    def _whole(a):
        return pl.BlockSpec(a.shape, lambda i: (0,) * a.ndim)

    def _rows(a):
        return pl.BlockSpec((_BLK, N), lambda i: (i, 0))

    wspecs_a = [kr(Wg0), rep(bg0), kr(Wg1), rep(bg1),
                kr(Wg2), rep(bg2), kr(Wg3), rep(bg3)]
    feat = pl.pallas_call(
        _chain_body,
        grid=(_NBLK,),
        in_specs=[_rows(L), _rows(A), _whole(x2), _whole(b2)]
                 + [_whole(w) for w in wspecs_a],
        out_specs=pl.BlockSpec((N, FD * B), lambda i: (0, 0)),
        out_shape=jax.ShapeDtypeStruct((N, FD * B), f32),
        scratch_shapes=[pltpu.VMEM((N, 64), f32), pltpu.VMEM((N, 64), f32),
                        pltpu.VMEM((N, 2 * N), jnp.bfloat16)],
    )(L, A, x2, b2, *wspecs_a)

    def ms(Ls, ft, W0, B0, W1, B1):
        return pl.pallas_call(
            _ms_body,
            grid=(_NBLK,),
            in_specs=[_rows(Ls), _whole(ft), _whole(W0), _whole(B0),
                      _whole(W1), _whole(B1)],
            out_specs=pl.BlockSpec((N, FD * B), lambda i: (0, 0)),
            out_shape=jax.ShapeDtypeStruct((N, FD * B), f32),
            scratch_shapes=[pltpu.VMEM((N, 24), f32),
                            pltpu.VMEM((N, 24), f32),
                            pltpu.VMEM((N, 2 * N), jnp.bfloat16)],
        )(Ls, ft, W0, B0, W1, B1)
    g2s = [ms(Ls, feat, kr(Wms0[s]), rep(bms0[s]), kr(Wms1[s]), rep(bms1[s]))
           for s, Ls in enumerate((L0, L1, L2, L3))]

    attn = pl.kernel(
        _attn_body,
        mesh=plsc.VectorSubcoreMesh(core_axis_name="c", subcore_axis_name="s"),
        compiler_params=pltpu.CompilerParams(needs_layout_passes=False),
        out_type=jax.ShapeDtypeStruct((32, _NPW * FD * B), f32),
        scratch_types=[
            pltpu.VMEM((N * FD * B,), f32),
            pltpu.VMEM((_NPW * K,), jnp.int32),
            pltpu.VMEM((_NPW * FD * B,), f32),
        ],
    )
    knn_flat = knn_idx.astype(jnp.int32).reshape(-1)
    aggs = [attn(g.reshape(-1), knn_flat).reshape(N, FD * B) for g in g2s]

    out2 = pl.pallas_call(
        _final_body,
        out_shape=jax.ShapeDtypeStruct((N, B), f32),
    )(aggs[0], aggs[1], aggs[2], aggs[3], sens_w.astype(f32), feat, x2,
      kr(Wp1), rep(bp1), kr(Wp2), rep(bp2), kr(Wu), rep(bu))

    return out2.T.reshape(B, N, 1)


# speedup vs baseline: 1.0759x; 1.0213x over previous
import jax
import jax.numpy as jnp
import numpy as np
from jax import lax
from jax.experimental import pallas as pl
from jax.experimental.pallas import tpu as pltpu
from jax.experimental.pallas import tpu_sc as plsc

N = 2048
B = 4
K = 16
FD = 6

_PREC = lax.Precision.HIGHEST


def _dot(a, bm):
    return jnp.dot(a, bm, preferred_element_type=jnp.float32, precision=_PREC)


def _relu(v):
    return jnp.maximum(v, 0.0)


_CHUNK = 1024
_NCH = N // _CHUNK
_BF = jnp.bfloat16


def _split(v):
    hi = v.astype(_BF)
    lo = (v - hi.astype(jnp.float32)).astype(_BF)
    return hi, lo


def _bdot(a, bm):
    return jnp.dot(a, bm, preferred_element_type=jnp.float32)


_BLK = 512
_NBLK = N // _BLK


def _rhs_cat(src_ref, cin):
    hh, hl = _split(src_ref[:, :cin])
    return jnp.concatenate(
        [jnp.concatenate([hh, hl], axis=1),
         jnp.concatenate([hh, jnp.zeros_like(hl)], axis=1)], axis=0)


def _mm_chunk(mcat, rhs, cin, W, bias):
    p = _bdot(mcat, rhs)
    t = p[:, :cin] + p[:, cin:2 * cin]
    return _relu(_dot(t, W) + bias)


def _layer(src_ref, cin, cout, Wr, Br, dst_ref, cat_ref):
    W = Wr[...]
    bias = Br[...]
    rhs = _rhs_cat(src_ref, cin)

    def chunk(i, carry):
        off = i * _CHUNK
        mcat = cat_ref[pl.ds(off, _CHUNK), :]
        dst_ref[pl.ds(off, _CHUNK), :cout] = _mm_chunk(mcat, rhs, cin, W,
                                                       bias)
        return carry

    lax.fori_loop(0, _NCH, chunk, jnp.int32(0))


def _chain_body(L_ref, A_ref, x2_ref, b2_ref, W0, B0, W1, B1, W2, B2, W3, B3,
                feat_ref, h_sc, t_sc, cat_sc):
    i = pl.program_id(0)
    off = i * _BLK
    x2 = x2_ref[...]

    @pl.when(i == 0)
    def _():
        h_sc[:, 0:4] = x2
        h_sc[:, 8:12] = b2_ref[...]

    xh, xl = _split(x2)
    ahi, alo = _split(A_ref[...])
    h_sc[pl.ds(off, _BLK), 4:8] = (
        _bdot(ahi, xh) + _bdot(ahi, xl) + _bdot(alo, xh))
    mhi, mlo = _split(L_ref[...])
    cat_sc[pl.ds(off, _BLK), 0:N] = mhi
    cat_sc[pl.ds(off, _BLK), N:2 * N] = mlo

    @pl.when(i == _NBLK - 1)
    def _():
        _layer(h_sc, 12, 32, W0, B0, t_sc, cat_sc)
        _layer(t_sc, 32, 64, W1, B1, h_sc, cat_sc)
        _layer(h_sc, 64, 32, W2, B2, t_sc, cat_sc)
        _layer(t_sc, 32, 24, W3, B3, h_sc, cat_sc)
        feat_ref[...] = h_sc[:, :24]


def _ms_body(L_ref, feat_ref, W0, B0, W1, B1, out_ref, g_sc, t_sc, cat_sc):
    i = pl.program_id(0)
    off = i * _BLK

    @pl.when(i == 0)
    def _():
        g_sc[:, :24] = feat_ref[...]

    mhi, mlo = _split(L_ref[...])
    cat_sc[pl.ds(off, _BLK), 0:N] = mhi
    cat_sc[pl.ds(off, _BLK), N:2 * N] = mlo
    rhs1 = _rhs_cat(g_sc, 24)
    t_sc[pl.ds(off, _BLK), :24] = _mm_chunk(
        jnp.concatenate([mhi, mlo], axis=1), rhs1, 24, W0[...], B0[...])

    @pl.when(i == _NBLK - 1)
    def _():
        _layer(t_sc, 24, 24, W1, B1, g_sc, cat_sc)
        out_ref[...] = g_sc[:, :24]


_NPW = N // 32


def _attn_body(gh, knnh, outh, g2_v, knn_v, out_v):
    cid = lax.axis_index("c")
    sid = lax.axis_index("s")
    wid = sid * 2 + cid
    base = wid * _NPW

    pltpu.sync_copy(gh, g2_v)
    pltpu.sync_copy(knnh.at[pl.ds(base * K, _NPW * K)], knn_v)

    iota16 = lax.iota(jnp.int32, 16)
    inv_sqrt = np.float32(1.0 / np.sqrt(float(FD)))
    CB = FD * B
    nj = _NPW // 16

    def body(i, carry):
        bb = i // nj
        j = i % nj
        lrow = j * 16 + iota16
        grow = base + lrow
        cols = [cc * 4 + bb for cc in range(FD)]
        h = [plsc.load_gather(g2_v, [grow * CB + cols[cc]])
             for cc in range(FD)]
        scr = []
        for k in range(K):
            idx = plsc.load_gather(knn_v, [lrow * K + k])
            nbase = idx * CB
            s_k = h[0] * plsc.load_gather(g2_v, [nbase + cols[0]])
            for cc in range(1, FD):
                s_k = s_k + h[cc] * plsc.load_gather(g2_v, [nbase + cols[cc]])
            scr.append(s_k * inv_sqrt)
        m = scr[0]
        for k in range(1, K):
            m = jnp.maximum(m, scr[k])
        ek = [jnp.exp(scr[k] - m) for k in range(K)]
        ssum = ek[0]
        for k in range(1, K):
            ssum = ssum + ek[k]
        inv = 1.0 / ssum
        att = [ek[k] * inv for k in range(K)]
        agg = [jnp.zeros((16,), jnp.float32) for _ in range(FD)]
        for k in range(K):
            idx = plsc.load_gather(knn_v, [lrow * K + k])
            nbase = idx * CB
            for cc in range(FD):
                nb = plsc.load_gather(g2_v, [nbase + cols[cc]])
                agg[cc] = agg[cc] + att[k] * nb
        for cc in range(FD):
            plsc.store_scatter(out_v, [lrow * CB + cols[cc]], agg[cc])
        return carry

    lax.fori_loop(0, B * nj, body, jnp.int32(0))
    pltpu.sync_copy(out_v, outh.at[wid])


def _final_body(a0, a1, a2, a3, sens_ref, feat_ref, x2_ref, Wp1r, bp1r,
                Wp2r, bp2r, Wur, bur, out_ref):
    sv = sens_ref[...]
    e = jnp.exp(sv - jnp.max(sv))
    w = e / jnp.sum(e)
    fused = w[0] * a0[...] + w[1] * a1[...] + w[2] * a2[...] + w[3] * a3[...]
    t1 = _relu(_dot(fused, Wp1r[...]) + bp1r[...])
    grad = _dot(t1, Wp2r[...]) + bp2r[...]
    alpha = jax.nn.sigmoid(_dot(feat_ref[...], Wur[...]) + bur[...])
    out_ref[...] = x2_ref[...] - alpha * grad


def kernel(x, b, L, A, L0, L1, L2, L3, knn_idx, sens_w, Wg0, bg0, Wg1, bg1,
           Wg2, bg2, Wg3, bg3, Wms0, bms0, Wms1, bms1, Wp1, bp1, Wp2, bp2,
           Wu, bu):
    f32 = jnp.float32
    eye = jnp.eye(B, dtype=f32)
    x2 = x[:, :, 0].T
    b2 = b[:, :, 0].T

    def kr(W):
        return jnp.kron(W.astype(f32), eye)

    def rep(v):
        return jnp.repeat(v.astype(f32), B)

    def _whole(a):
        return pl.BlockSpec(a.shape, lambda i: (0,) * a.ndim)

    def _rows(a):
        return pl.BlockSpec((_BLK, N), lambda i: (i, 0))

    wspecs_a = [kr(Wg0), rep(bg0), kr(Wg1), rep(bg1),
                kr(Wg2), rep(bg2), kr(Wg3), rep(bg3)]
    feat = pl.pallas_call(
        _chain_body,
        grid=(_NBLK,),
        in_specs=[_rows(L), _rows(A), _whole(x2), _whole(b2)]
                 + [_whole(w) for w in wspecs_a],
        out_specs=pl.BlockSpec((N, FD * B), lambda i: (0, 0)),
        out_shape=jax.ShapeDtypeStruct((N, FD * B), f32),
        scratch_shapes=[pltpu.VMEM((N, 64), f32), pltpu.VMEM((N, 64), f32),
                        pltpu.VMEM((N, 2 * N), jnp.bfloat16)],
    )(L, A, x2, b2, *wspecs_a)

    def ms(Ls, ft, W0, B0, W1, B1):
        return pl.pallas_call(
            _ms_body,
            grid=(_NBLK,),
            in_specs=[_rows(Ls), _whole(ft), _whole(W0), _whole(B0),
                      _whole(W1), _whole(B1)],
            out_specs=pl.BlockSpec((N, FD * B), lambda i: (0, 0)),
            out_shape=jax.ShapeDtypeStruct((N, FD * B), f32),
            scratch_shapes=[pltpu.VMEM((N, 24), f32),
                            pltpu.VMEM((N, 24), f32),
                            pltpu.VMEM((N, 2 * N), jnp.bfloat16)],
        )(Ls, ft, W0, B0, W1, B1)
    g2s = [ms(Ls, feat, kr(Wms0[s]), rep(bms0[s]), kr(Wms1[s]), rep(bms1[s]))
           for s, Ls in enumerate((L0, L1, L2, L3))]

    attn = pl.kernel(
        _attn_body,
        mesh=plsc.VectorSubcoreMesh(core_axis_name="c", subcore_axis_name="s"),
        compiler_params=pltpu.CompilerParams(needs_layout_passes=False),
        out_type=jax.ShapeDtypeStruct((32, _NPW * FD * B), f32),
        scratch_types=[
            pltpu.VMEM((N * FD * B,), f32),
            pltpu.VMEM((_NPW * K,), jnp.int32),
            pltpu.VMEM((_NPW * FD * B,), f32),
        ],
    )
    knn_flat = knn_idx.astype(jnp.int32).reshape(-1)
    aggs = [attn(g.reshape(-1), knn_flat).reshape(N, FD * B) for g in g2s]

    out2 = pl.pallas_call(
        _final_body,
        out_shape=jax.ShapeDtypeStruct((N, B), f32),
    )(aggs[0], aggs[1], aggs[2], aggs[3], sens_w.astype(f32), feat, x2,
      kr(Wp1), rep(bp1), kr(Wp2), rep(bp2), kr(Wu), rep(bu))

    return out2.T.reshape(B, N, 1)
